# baseline (speedup 1.0000x reference)
import functools

import jax
import jax.numpy as jnp
from jax import lax
from jax.experimental import pallas as pl
from jax.experimental.pallas import tpu as pltpu
from jax.experimental.pallas import tpu_sc as plsc

_N = 10000
_E = 320000
_D = 128
_G = 64

_NC = 2
_NS = 16
_NW = _NC * _NS
_EW = _E // _NW
_K = 128
_NCH = 81
_EWP = _NCH * _K
_NPAD = 10112
_RPT = _NPAD // _NS
_ZR = _K

_B = 1000
_GRID = _N // _B


def _msg_body(h, edges3, zrows, out, ib0, ib1, ib2, ib3, rows0, rows1,
              is0, is1, is2, is3, rs0, rs1, acc):
    c = lax.axis_index("c")
    s = lax.axis_index("s")
    wid = s * _NC + c
    cbase = wid * _NCH

    ibufs = (ib0, ib1, ib2, ib3)
    isems = (is0, is1, is2, is3)

    pltpu.sync_copy(zrows, rows0)
    for j in range(_RPT // _K):
        pltpu.sync_copy(rows0, acc.at[pl.ds(s * _RPT + j * _K, _K)])
    rem = _RPT - (_RPT // _K) * _K
    if rem:
        pltpu.sync_copy(rows0.at[pl.ds(0, rem)],
                        acc.at[pl.ds(s * _RPT + (_RPT // _K) * _K, rem)])
    plsc.subcore_barrier()

    def ifetch(i, t):
        @pl.when(i < _NCH)
        def _():
            pltpu.async_copy(edges3.at[cbase + i], ibufs[t], isems[t])

    def iwait(t):
        pltpu.make_async_copy(edges3.at[cbase], ibufs[t], isems[t]).wait()

    def gstart(t, rows, sem):
        pltpu.async_copy(h.at[ibufs[t].at[0]], rows, sem)

    def gwait(rows, sem):
        pltpu.make_async_copy(h.at[ibufs[0].at[0]], rows, sem).wait()

    def scat(t, rows):
        pltpu.sync_copy(rows, acc.at[ibufs[t].at[1]], add=True)

    for t in range(4):
        ifetch(t, t)
    iwait(0)
    gstart(0, rows0, rs0)

    def quad(j, carry):
        c0 = 4 * j
        iwait(1)
        gstart(1, rows1, rs1)
        gwait(rows0, rs0)
        scat(0, rows0)
        ifetch(c0 + 4, 0)
        iwait(2)
        gstart(2, rows0, rs0)
        gwait(rows1, rs1)
        scat(1, rows1)
        ifetch(c0 + 5, 1)
        iwait(3)
        gstart(3, rows1, rs1)
        gwait(rows0, rs0)
        scat(2, rows0)
        ifetch(c0 + 6, 2)
        iwait(0)
        gstart(0, rows0, rs0)
        gwait(rows1, rs1)
        scat(3, rows1)
        ifetch(c0 + 7, 3)
        return carry

    lax.fori_loop(0, (_NCH - 1) // 4, quad, 0)
    gwait(rows0, rs0)
    scat(0, rows0)

    plsc.subcore_barrier()
    pltpu.sync_copy(acc.at[pl.ds(s * _RPT, _RPT)],
                    out.at[c, pl.ds(s * _RPT, _RPT)])


@functools.cache
def _msg_kernel():
    return pl.kernel(
        _msg_body,
        out_type=jax.ShapeDtypeStruct((_NC, _NPAD, _D), jnp.float32),
        mesh=plsc.VectorSubcoreMesh(core_axis_name="c", subcore_axis_name="s"),
        scratch_types=(
            [pltpu.VMEM((2, _K), jnp.int32)] * 4
            + [pltpu.VMEM((_K, _D), jnp.float32)] * 2
            + [pltpu.SemaphoreType.DMA] * 6
            + [pltpu.VMEM_SHARED((_NPAD, _D), jnp.float32)]
        ),
    )


def _mlp_body(h, a0, a1, wa, ba, wb, bb, out, *, relu_out):
    z = h[...] + a0[...] + a1[...]
    u = jnp.maximum(
        jnp.dot(z, wa[...], preferred_element_type=jnp.float32) + ba[...], 0.0)
    v = jnp.dot(u, wb[...], preferred_element_type=jnp.float32) + bb[...]
    if relu_out:
        v = jnp.maximum(v, 0.0)
    out[...] = v


def _mlp(h, a0, a1, waT, ba, wbT, bb, relu_out):
    row = pl.BlockSpec((_B, _D), lambda i: (i, 0))
    full = pl.BlockSpec((_D, _D), lambda i: (0, 0))
    bias = pl.BlockSpec((1, _D), lambda i: (0, 0))
    return pl.pallas_call(
        functools.partial(_mlp_body, relu_out=relu_out),
        grid=(_GRID,),
        in_specs=[row, row, row, full, bias, full, bias],
        out_specs=row,
        out_shape=jax.ShapeDtypeStruct((_N, _D), jnp.float32),
    )(h, a0, a1, waT, ba.reshape(1, _D), wbT, bb.reshape(1, _D))


def _mlp_pool_body(h, a0, a1, wa, ba, wb, bb, batch, out, accs, accc):
    i = pl.program_id(0)

    @pl.when(i == 0)
    def _init():
        accs[...] = jnp.zeros_like(accs)
        accc[...] = jnp.zeros_like(accc)

    z = h[...] + a0[...] + a1[...]
    u = jnp.maximum(
        jnp.dot(z, wa[...], preferred_element_type=jnp.float32) + ba[...], 0.0)
    v = jnp.dot(u, wb[...], preferred_element_type=jnp.float32) + bb[...]

    b = batch[...].reshape(1, _B)
    onehot = (b == lax.broadcasted_iota(jnp.int32, (_G, _B), 0)).astype(jnp.float32)
    accs[...] += jnp.dot(onehot, v, preferred_element_type=jnp.float32)
    accc[...] += jnp.sum(onehot, axis=1, keepdims=True)

    @pl.when(i == _GRID - 1)
    def _fin():
        out[...] = accs[...] / jnp.maximum(accc[...], 1.0)


def _mlp_pool(h, a0, a1, waT, ba, wbT, bb, batch3):
    row = pl.BlockSpec((_B, _D), lambda i: (i, 0))
    full = pl.BlockSpec((_D, _D), lambda i: (0, 0))
    bias = pl.BlockSpec((1, _D), lambda i: (0, 0))
    bspec = pl.BlockSpec((1, 1, _B), lambda i: (i, 0, 0))
    return pl.pallas_call(
        _mlp_pool_body,
        grid=(_GRID,),
        in_specs=[row, row, row, full, bias, full, bias, bspec],
        out_specs=pl.BlockSpec((_G, _D), lambda i: (0, 0)),
        out_shape=jax.ShapeDtypeStruct((_G, _D), jnp.float32),
        scratch_shapes=[
            pltpu.VMEM((_G, _D), jnp.float32),
            pltpu.VMEM((_G, 1), jnp.float32),
        ],
    )(h, a0, a1, waT, ba.reshape(1, _D), wbT, bb.reshape(1, _D), batch3)


def kernel(x, edge_index, batch, W0a, b0a, W0b, b0b, W1a, b1a, W1b, b1b,
           W2a, b2a, W2b, b2b):
    ei = edge_index.reshape(2, _NW, _EW)
    pad = jnp.stack([jnp.zeros((_NW, _EWP - _EW), jnp.int32),
                     jnp.full((_NW, _EWP - _EW), _N, jnp.int32)])
    edges3 = jnp.transpose(
        jnp.concatenate([ei, pad], axis=2).reshape(2, _NW, _NCH, _K),
        (1, 2, 0, 3)).reshape(_NW * _NCH, 2, _K)
    zrows = jnp.zeros((_K, _D), jnp.float32)
    batch3 = batch.reshape(_GRID, 1, _B)
    _msg_raw = _msg_kernel()

    def msg(h):
        a = _msg_raw(h, edges3, zrows)
        return a[:, :_N, :]

    h = x
    agg = msg(h)
    h = _mlp(h, agg[0], agg[1], W0a.T, b0a, W0b.T, b0b, relu_out=True)
    agg = msg(h)
    h = _mlp(h, agg[0], agg[1], W1a.T, b1a, W1b.T, b1b, relu_out=True)
    agg = msg(h)
    return _mlp_pool(h, agg[0], agg[1], W2a.T, b2a, W2b.T, b2b, batch3)


# speedup vs baseline: 1.0011x; 1.0011x over previous
import functools

import jax
import jax.numpy as jnp
from jax import lax
from jax.experimental import pallas as pl
from jax.experimental.pallas import tpu as pltpu
from jax.experimental.pallas import tpu_sc as plsc

_N = 10000
_E = 320000
_D = 128
_G = 64

_NC = 2
_NS = 16
_NW = _NC * _NS
_EW = _E // _NW
_K = 128
_NCH = 81
_EWP = _NCH * _K
_NPAD = 10112
_RPT = _NPAD // _NS
_ZR = _K

_B = 1000
_GRID = _N // _B


def _msg_body(h, edges3, zrows, out, ib0, ib1, ib2, ib3, rows0, rows1,
              is0, is1, is2, is3, rs0, rs1, acc):
    c = lax.axis_index("c")
    s = lax.axis_index("s")
    wid = s * _NC + c
    cbase = wid * _NCH

    ibufs = (ib0, ib1, ib2, ib3)
    isems = (is0, is1, is2, is3)

    pltpu.sync_copy(zrows, rows0)
    for j in range(_RPT // _K):
        pltpu.sync_copy(rows0, acc.at[pl.ds(s * _RPT + j * _K, _K)])
    rem = _RPT - (_RPT // _K) * _K
    if rem:
        pltpu.sync_copy(rows0.at[pl.ds(0, rem)],
                        acc.at[pl.ds(s * _RPT + (_RPT // _K) * _K, rem)])
    plsc.subcore_barrier()

    def ifetch(i, t):
        @pl.when(i < _NCH)
        def _():
            pltpu.async_copy(edges3.at[cbase + i], ibufs[t], isems[t])

    def iwait(t):
        pltpu.make_async_copy(edges3.at[cbase], ibufs[t], isems[t]).wait()

    def gstart(t, rows, sem):
        pltpu.async_copy(h.at[ibufs[t].at[0]], rows, sem)

    def gwait(rows, sem):
        pltpu.make_async_copy(h.at[ibufs[0].at[0]], rows, sem).wait()

    def scat(t, rows):
        pltpu.sync_copy(rows, acc.at[ibufs[t].at[1]], add=True)

    for t in range(4):
        ifetch(t, t)
    iwait(0)
    gstart(0, rows0, rs0)

    def quad(j, carry):
        c0 = 4 * j
        iwait(1)
        gstart(1, rows1, rs1)
        gwait(rows0, rs0)
        scat(0, rows0)
        ifetch(c0 + 4, 0)
        iwait(2)
        gstart(2, rows0, rs0)
        gwait(rows1, rs1)
        scat(1, rows1)
        ifetch(c0 + 5, 1)
        iwait(3)
        gstart(3, rows1, rs1)
        gwait(rows0, rs0)
        scat(2, rows0)
        ifetch(c0 + 6, 2)
        iwait(0)
        gstart(0, rows0, rs0)
        gwait(rows1, rs1)
        scat(3, rows1)
        ifetch(c0 + 7, 3)
        return carry

    lax.fori_loop(0, (_NCH - 1) // 4, quad, 0)
    gwait(rows0, rs0)
    scat(0, rows0)

    plsc.subcore_barrier()
    pltpu.sync_copy(acc.at[pl.ds(s * _RPT, _RPT)],
                    out.at[c, pl.ds(s * _RPT, _RPT)])


@functools.cache
def _msg_kernel():
    return pl.kernel(
        _msg_body,
        out_type=jax.ShapeDtypeStruct((_NC, _NPAD, _D), jnp.float32),
        mesh=plsc.VectorSubcoreMesh(core_axis_name="c", subcore_axis_name="s"),
        scratch_types=(
            [pltpu.VMEM((2, _K), jnp.int32)] * 4
            + [pltpu.VMEM((_K, _D), jnp.float32)] * 2
            + [pltpu.SemaphoreType.DMA] * 6
            + [pltpu.VMEM_SHARED((_NPAD, _D), jnp.float32)]
        ),
    )


def _mlp_body(h, a0, a1, wa, ba, wb, bb, out, *, relu_out):
    z = h[...] + a0[...] + a1[...]
    u = jnp.maximum(
        jnp.dot(z, wa[...], preferred_element_type=jnp.float32) + ba[...], 0.0)
    v = jnp.dot(u, wb[...], preferred_element_type=jnp.float32) + bb[...]
    if relu_out:
        v = jnp.maximum(v, 0.0)
    out[...] = v


def _mlp(h, a0, a1, waT, ba, wbT, bb, relu_out):
    row = pl.BlockSpec((_B, _D), lambda i: (i, 0))
    full = pl.BlockSpec((_D, _D), lambda i: (0, 0))
    bias = pl.BlockSpec((1, _D), lambda i: (0, 0))
    return pl.pallas_call(
        functools.partial(_mlp_body, relu_out=relu_out),
        grid=(_GRID,),
        in_specs=[row, row, row, full, bias, full, bias],
        out_specs=row,
        out_shape=jax.ShapeDtypeStruct((_N, _D), jnp.float32),
    )(h, a0, a1, waT, ba.reshape(1, _D), wbT, bb.reshape(1, _D))


def _mlp_pool_body(h, a0, a1, wa, ba, wb, bb, batch, out, accs, accc):
    i = pl.program_id(0)

    @pl.when(i == 0)
    def _init():
        accs[...] = jnp.zeros_like(accs)
        accc[...] = jnp.zeros_like(accc)

    z = h[...] + a0[...] + a1[...]
    u = jnp.maximum(
        jnp.dot(z, wa[...], preferred_element_type=jnp.float32) + ba[...], 0.0)
    v = jnp.dot(u, wb[...], preferred_element_type=jnp.float32) + bb[...]

    b = batch[...].reshape(1, _B)
    onehot = (b == lax.broadcasted_iota(jnp.int32, (_G, _B), 0)).astype(jnp.float32)
    accs[...] += jnp.dot(onehot, v, preferred_element_type=jnp.float32)
    accc[...] += jnp.sum(onehot, axis=1, keepdims=True)

    @pl.when(i == _GRID - 1)
    def _fin():
        out[...] = accs[...] / jnp.maximum(accc[...], 1.0)


def _mlp_pool(h, a0, a1, waT, ba, wbT, bb, batch3):
    row = pl.BlockSpec((_B, _D), lambda i: (i, 0))
    full = pl.BlockSpec((_D, _D), lambda i: (0, 0))
    bias = pl.BlockSpec((1, _D), lambda i: (0, 0))
    bspec = pl.BlockSpec((1, 1, _B), lambda i: (i, 0, 0))
    return pl.pallas_call(
        _mlp_pool_body,
        grid=(_GRID,),
        in_specs=[row, row, row, full, bias, full, bias, bspec],
        out_specs=pl.BlockSpec((_G, _D), lambda i: (0, 0)),
        out_shape=jax.ShapeDtypeStruct((_G, _D), jnp.float32),
        scratch_shapes=[
            pltpu.VMEM((_G, _D), jnp.float32),
            pltpu.VMEM((_G, 1), jnp.float32),
        ],
    )(h, a0, a1, waT, ba.reshape(1, _D), wbT, bb.reshape(1, _D), batch3)


def kernel(x, edge_index, batch, W0a, b0a, W0b, b0b, W1a, b1a, W1b, b1b,
           W2a, b2a, W2b, b2b):
    ei = edge_index.reshape(2, _NW, _EW)
    npd = _EWP - _EW
    pad_dst = _N + (jnp.arange(npd, dtype=jnp.int32)[None, :]
                    + 7 * jnp.arange(_NW, dtype=jnp.int32)[:, None]) % (_NPAD - _N)
    pad = jnp.stack([jnp.zeros((_NW, npd), jnp.int32), pad_dst])
    edges3 = jnp.transpose(
        jnp.concatenate([ei, pad], axis=2).reshape(2, _NW, _NCH, _K),
        (1, 2, 0, 3)).reshape(_NW * _NCH, 2, _K)
    zrows = jnp.zeros((_K, _D), jnp.float32)
    batch3 = batch.reshape(_GRID, 1, _B)
    _msg_raw = _msg_kernel()

    def msg(h):
        a = _msg_raw(h, edges3, zrows)
        return a[:, :_N, :]

    h = x
    agg = msg(h)
    h = _mlp(h, agg[0], agg[1], W0a.T, b0a, W0b.T, b0b, relu_out=True)
    agg = msg(h)
    h = _mlp(h, agg[0], agg[1], W1a.T, b1a, W1b.T, b1b, relu_out=True)
    agg = msg(h)
    return _mlp_pool(h, agg[0], agg[1], W2a.T, b2a, W2b.T, b2b, batch3)


# speedup vs baseline: 2.4710x; 2.4683x over previous
import functools

import jax
import jax.numpy as jnp
from jax import lax
from jax.experimental import pallas as pl
from jax.experimental.pallas import tpu as pltpu
from jax.experimental.pallas import tpu_sc as plsc

_N = 10000
_E = 320000
_D = 128
_G = 64

_NC = 2
_NS = 16
_NW = _NC * _NS
_EW = _E // _NW
_K = 96
_NCH = 105
_EWP = _NCH * _K
_NPAD = 10112
_RPT = _NPAD // _NS
_ZR = _K

_B = 1000
_GRID = _N // _B


def _msg_body(h, edges3, zrows, out, ib0, ib1, ib2, ib3, rows0, rows1,
              is0, is1, is2, is3, rs0, rs1, ss0, ss1, acc):
    c = lax.axis_index("c")
    s = lax.axis_index("s")
    wid = s * _NC + c
    cbase = wid * _NCH

    ibufs = (ib0, ib1, ib2, ib3)
    isems = (is0, is1, is2, is3)

    pltpu.sync_copy(zrows, rows0)
    for j in range(_RPT // _K):
        pltpu.sync_copy(rows0, acc.at[pl.ds(s * _RPT + j * _K, _K)])
    rem = _RPT - (_RPT // _K) * _K
    if rem:
        pltpu.sync_copy(rows0.at[pl.ds(0, rem)],
                        acc.at[pl.ds(s * _RPT + (_RPT // _K) * _K, rem)])
    plsc.subcore_barrier()

    def ifetch(i, t):
        @pl.when(i < _NCH)
        def _():
            pltpu.async_copy(edges3.at[cbase + i], ibufs[t], isems[t])

    def iwait(t):
        pltpu.make_async_copy(edges3.at[cbase], ibufs[t], isems[t]).wait()

    def gstart(t, rows, sem):
        pltpu.async_copy(h.at[ibufs[t].at[0]], rows, sem)

    def gwait(rows, sem):
        pltpu.make_async_copy(h.at[ibufs[0].at[0]], rows, sem).wait()

    def sstart(t, rows, sem):
        pltpu.async_copy(rows, acc.at[ibufs[t].at[1]], sem, add=True)

    def swait(rows, sem):
        pltpu.make_async_copy(rows, acc.at[ibufs[0].at[1]], sem).wait()

    for t in range(4):
        ifetch(t, t)
    iwait(0)
    gstart(0, rows0, rs0)

    def quad_steps(c0, first):
        iwait(1)
        if not first:
            swait(rows1, ss1)
        gstart(1, rows1, rs1)
        gwait(rows0, rs0)
        sstart(0, rows0, ss0)
        ifetch(c0 + 4, 0)
        iwait(2)
        swait(rows0, ss0)
        gstart(2, rows0, rs0)
        gwait(rows1, rs1)
        sstart(1, rows1, ss1)
        ifetch(c0 + 5, 1)
        iwait(3)
        swait(rows1, ss1)
        gstart(3, rows1, rs1)
        gwait(rows0, rs0)
        sstart(2, rows0, ss0)
        ifetch(c0 + 6, 2)
        iwait(0)
        swait(rows0, ss0)
        gstart(0, rows0, rs0)
        gwait(rows1, rs1)
        sstart(3, rows1, ss1)
        ifetch(c0 + 7, 3)

    def quad(j, carry):
        quad_steps(4 * j, False)
        return carry

    quad_steps(0, True)
    lax.fori_loop(1, (_NCH - 1) // 4, quad, 0)
    gwait(rows0, rs0)
    sstart(0, rows0, ss0)
    swait(rows0, ss0)
    swait(rows1, ss1)

    plsc.subcore_barrier()
    pltpu.sync_copy(acc.at[pl.ds(s * _RPT, _RPT)],
                    out.at[c, pl.ds(s * _RPT, _RPT)])


@functools.cache
def _msg_kernel():
    return pl.kernel(
        _msg_body,
        out_type=jax.ShapeDtypeStruct((_NC, _NPAD, _D), jnp.float32),
        mesh=plsc.VectorSubcoreMesh(core_axis_name="c", subcore_axis_name="s"),
        scratch_types=(
            [pltpu.VMEM((2, _K), jnp.int32)] * 4
            + [pltpu.VMEM((_K, _D), jnp.float32)] * 2
            + [pltpu.SemaphoreType.DMA] * 8
            + [pltpu.VMEM_SHARED((_NPAD, _D), jnp.float32)]
        ),
    )


def _mlp_body(h, a0, a1, wa, ba, wb, bb, out, *, relu_out):
    z = h[...] + a0[...] + a1[...]
    u = jnp.maximum(
        jnp.dot(z, wa[...], preferred_element_type=jnp.float32) + ba[...], 0.0)
    v = jnp.dot(u, wb[...], preferred_element_type=jnp.float32) + bb[...]
    if relu_out:
        v = jnp.maximum(v, 0.0)
    out[...] = v


def _mlp(h, a0, a1, waT, ba, wbT, bb, relu_out):
    row = pl.BlockSpec((_B, _D), lambda i: (i, 0))
    full = pl.BlockSpec((_D, _D), lambda i: (0, 0))
    bias = pl.BlockSpec((1, _D), lambda i: (0, 0))
    return pl.pallas_call(
        functools.partial(_mlp_body, relu_out=relu_out),
        grid=(_GRID,),
        in_specs=[row, row, row, full, bias, full, bias],
        out_specs=row,
        out_shape=jax.ShapeDtypeStruct((_N, _D), jnp.float32),
    )(h, a0, a1, waT, ba.reshape(1, _D), wbT, bb.reshape(1, _D))


def _mlp_pool_body(h, a0, a1, wa, ba, wb, bb, batch, out, accs, accc):
    i = pl.program_id(0)

    @pl.when(i == 0)
    def _init():
        accs[...] = jnp.zeros_like(accs)
        accc[...] = jnp.zeros_like(accc)

    z = h[...] + a0[...] + a1[...]
    u = jnp.maximum(
        jnp.dot(z, wa[...], preferred_element_type=jnp.float32) + ba[...], 0.0)
    v = jnp.dot(u, wb[...], preferred_element_type=jnp.float32) + bb[...]

    b = batch[...].reshape(1, _B)
    onehot = (b == lax.broadcasted_iota(jnp.int32, (_G, _B), 0)).astype(jnp.float32)
    accs[...] += jnp.dot(onehot, v, preferred_element_type=jnp.float32)
    accc[...] += jnp.sum(onehot, axis=1, keepdims=True)

    @pl.when(i == _GRID - 1)
    def _fin():
        out[...] = accs[...] / jnp.maximum(accc[...], 1.0)


def _mlp_pool(h, a0, a1, waT, ba, wbT, bb, batch3):
    row = pl.BlockSpec((_B, _D), lambda i: (i, 0))
    full = pl.BlockSpec((_D, _D), lambda i: (0, 0))
    bias = pl.BlockSpec((1, _D), lambda i: (0, 0))
    bspec = pl.BlockSpec((1, 1, _B), lambda i: (i, 0, 0))
    return pl.pallas_call(
        _mlp_pool_body,
        grid=(_GRID,),
        in_specs=[row, row, row, full, bias, full, bias, bspec],
        out_specs=pl.BlockSpec((_G, _D), lambda i: (0, 0)),
        out_shape=jax.ShapeDtypeStruct((_G, _D), jnp.float32),
        scratch_shapes=[
            pltpu.VMEM((_G, _D), jnp.float32),
            pltpu.VMEM((_G, 1), jnp.float32),
        ],
    )(h, a0, a1, waT, ba.reshape(1, _D), wbT, bb.reshape(1, _D), batch3)


def kernel(x, edge_index, batch, W0a, b0a, W0b, b0b, W1a, b1a, W1b, b1b,
           W2a, b2a, W2b, b2b):
    ei = edge_index.reshape(2, _NW, _EW)
    npd = _EWP - _EW
    pad_dst = _N + (jnp.arange(npd, dtype=jnp.int32)[None, :]
                    + 7 * jnp.arange(_NW, dtype=jnp.int32)[:, None]) % (_NPAD - _N)
    pad = jnp.stack([jnp.zeros((_NW, npd), jnp.int32), pad_dst])
    edges3 = jnp.transpose(
        jnp.concatenate([ei, pad], axis=2).reshape(2, _NW, _NCH, _K),
        (1, 2, 0, 3)).reshape(_NW * _NCH, 2, _K)
    zrows = jnp.zeros((_K, _D), jnp.float32)
    batch3 = batch.reshape(_GRID, 1, _B)
    _msg_raw = _msg_kernel()

    def msg(h):
        a = _msg_raw(h, edges3, zrows)
        return a[:, :_N, :]

    h = x
    agg = msg(h)
    h = _mlp(h, agg[0], agg[1], W0a.T, b0a, W0b.T, b0b, relu_out=True)
    agg = msg(h)
    h = _mlp(h, agg[0], agg[1], W1a.T, b1a, W1b.T, b1b, relu_out=True)
    agg = msg(h)
    return _mlp_pool(h, agg[0], agg[1], W2a.T, b2a, W2b.T, b2b, batch3)
